# baseline (speedup 1.0000x reference)
import jax
import jax.numpy as jnp
from jax import lax
from jax.experimental import pallas as pl
from jax.experimental.pallas import tpu as pltpu
from jax.experimental.pallas import tpu_sc as plsc

B = 4096
G = 4
L = 50
D = 128
V = 10000
NW = 32
ROWS_PER_W = B // NW
IDX_PER_W = ROWS_PER_W * G * L
PAIR = 2 * G * L
NVEC = PAIR // 16


def _tc_project(b_ref, w_ref, table_ref, x_ref, t2_ref, x2_ref):
    @pl.when(pl.program_id(0) == 0)
    def _():
        t2_ref[...] = lax.dot_general(
            w_ref[...], table_ref[...],
            (((1,), (1,)), ((), ())),
            preferred_element_type=jnp.float32,
        ) * (1.0 / L) + b_ref[0] * (1.0 / (G * L))

    for g in range(G):
        x2_ref[0, pl.ds(g * L, L), :] = x_ref[:, g, :].T


def _sc_pool(idx_hbm, t2_hbm, out_hbm, idx_v, t2_v, out_v, sem_a, sem_b):
    wid = lax.axis_index("s") * 2 + lax.axis_index("c")
    base_row = wid * ROWS_PER_W
    cp_idx = pltpu.async_copy(idx_hbm.at[wid], idx_v, sem_a)
    cp_t2 = pltpu.async_copy(t2_hbm, t2_v, sem_b)
    cp_idx.wait()
    cp_t2.wait()

    def rg_body(rg, _):
        col = rg * 16

        def g_loop(acc, g):
            gv = jnp.full((16,), g, jnp.int32)

            def l_loop(l, acc):
                iv = idx_v[g * L + l, pl.ds(col, 16)]
                return acc + plsc.load_gather(t2_v, [gv, iv])

            return lax.fori_loop(0, L, l_loop, acc)

        acc = jnp.zeros((16,), jnp.float32)
        for g in range(G):
            acc = g_loop(acc, g)
        out_v[pl.ds(col, 16)] = acc
        return 0

    lax.fori_loop(0, ROWS_PER_W // 16, rg_body, 0)
    pltpu.sync_copy(out_v, out_hbm.at[pl.ds(base_row, ROWS_PER_W)])


def kernel(x, table, W, b):
    w2 = W.reshape(G, D)
    t2, x2 = pl.pallas_call(
        _tc_project,
        grid=(NW,),
        in_specs=[
            pl.BlockSpec(memory_space=pltpu.SMEM),
            pl.BlockSpec((G, D), lambda i: (0, 0)),
            pl.BlockSpec((V, D), lambda i: (0, 0)),
            pl.BlockSpec((ROWS_PER_W, G, L), lambda i: (i, 0, 0)),
        ],
        out_specs=[
            pl.BlockSpec((G, V), lambda i: (0, 0)),
            pl.BlockSpec((1, G * L, ROWS_PER_W), lambda i: (i, 0, 0)),
        ],
        out_shape=[
            jax.ShapeDtypeStruct((G, V), jnp.float32),
            jax.ShapeDtypeStruct((NW, G * L, ROWS_PER_W), jnp.int32),
        ],
    )(b, w2, table, x.astype(jnp.int32))

    sc = pl.kernel(
        _sc_pool,
        out_type=jax.ShapeDtypeStruct((B,), jnp.float32),
        mesh=plsc.VectorSubcoreMesh(core_axis_name="c", subcore_axis_name="s"),
        compiler_params=pltpu.CompilerParams(needs_layout_passes=False),
        scratch_types=[
            pltpu.VMEM((G * L, ROWS_PER_W), jnp.int32),
            pltpu.VMEM((G, V), jnp.float32),
            pltpu.VMEM((ROWS_PER_W,), jnp.float32),
            pltpu.SemaphoreType.DMA,
            pltpu.SemaphoreType.DMA,
        ],
    )
    out = sc(x2, t2)
    return out.reshape(B, 1)


# speedup vs baseline: 1.2304x; 1.2304x over previous
import jax
import jax.numpy as jnp
from jax import lax
from jax.experimental import pallas as pl
from jax.experimental.pallas import tpu as pltpu
from jax.experimental.pallas import tpu_sc as plsc

B = 4096
G = 4
L = 50
D = 128
V = 10000
NW = 32
ROWS_PER_W = B // NW


def _tc_project(b_ref, w_ref, table_ref, out_ref):
    out_ref[...] = lax.dot_general(
        w_ref[...], table_ref[...],
        (((1,), (1,)), ((), ())),
        preferred_element_type=jnp.float32,
    ) * (1.0 / L) + b_ref[0] * (1.0 / (G * L))


def _sc_pool(x_hbm, t2_hbm, out_hbm, idx_v, t2_v, out_v, sem_a, sem_b):
    wid = lax.axis_index("s") * 2 + lax.axis_index("c")
    base_row = wid * ROWS_PER_W
    cp_idx = pltpu.async_copy(x_hbm.at[pl.ds(base_row, ROWS_PER_W)], idx_v,
                              sem_a)
    cp_t2 = pltpu.async_copy(t2_hbm, t2_v, sem_b)
    lanes = lax.iota(jnp.int32, 16)
    cp_idx.wait()
    cp_t2.wait()

    def rg_body(rg, _):
        rv = lanes + rg * 16
        acc = jnp.zeros((16,), jnp.float32)
        for g in range(G):
            gv = jnp.full((16,), g, jnp.int32)
            for l in range(L):
                lv = jnp.full((16,), l, jnp.int32)
                iv = plsc.load_gather(idx_v, [rv, gv, lv])
                acc = acc + plsc.load_gather(t2_v, [gv, iv])
        out_v[pl.ds(rg * 16, 16)] = acc
        return 0

    lax.fori_loop(0, ROWS_PER_W // 16, rg_body, 0)
    pltpu.sync_copy(out_v, out_hbm.at[pl.ds(base_row, ROWS_PER_W)])


def kernel(x, table, W, b):
    w2 = W.reshape(G, D)
    t2 = pl.pallas_call(
        _tc_project,
        in_specs=[
            pl.BlockSpec(memory_space=pltpu.SMEM),
            pl.BlockSpec(memory_space=pltpu.VMEM),
            pl.BlockSpec(memory_space=pltpu.VMEM),
        ],
        out_shape=jax.ShapeDtypeStruct((G, V), jnp.float32),
    )(b, w2, table)

    sc = pl.kernel(
        _sc_pool,
        out_type=jax.ShapeDtypeStruct((B,), jnp.float32),
        mesh=plsc.VectorSubcoreMesh(core_axis_name="c", subcore_axis_name="s"),
        compiler_params=pltpu.CompilerParams(needs_layout_passes=False),
        scratch_types=[
            pltpu.VMEM((ROWS_PER_W, G, L), jnp.int32),
            pltpu.VMEM((G, V), jnp.float32),
            pltpu.VMEM((ROWS_PER_W,), jnp.float32),
            pltpu.SemaphoreType.DMA,
            pltpu.SemaphoreType.DMA,
        ],
    )
    out = sc(x.astype(jnp.int32), t2)
    return out.reshape(B, 1)


# speedup vs baseline: 1.4253x; 1.1584x over previous
import jax
import jax.numpy as jnp
from jax import lax
from jax.experimental import pallas as pl
from jax.experimental.pallas import tpu as pltpu
from jax.experimental.pallas import tpu_sc as plsc

B = 4096
G = 4
L = 50
D = 128
V = 10000
NW = 32
ROWS_PER_W = B // NW


def _tc_project(b_ref, w_ref, table_ref, out_ref):
    out_ref[...] = lax.dot_general(
        w_ref[...], table_ref[...],
        (((1,), (1,)), ((), ())),
        preferred_element_type=jnp.float32,
    ) * (1.0 / L) + b_ref[0] * (1.0 / (G * L))


def _sc_pool(x_hbm, t2_hbm, out_hbm, idx_v, t2_v, lvt_v, out_v, sem_a, sem_b):
    wid = lax.axis_index("s") * 2 + lax.axis_index("c")
    base_row = wid * ROWS_PER_W
    cp_idx = pltpu.async_copy(x_hbm.at[pl.ds(base_row, ROWS_PER_W)], idx_v,
                              sem_a)
    cp_t2 = pltpu.async_copy(t2_hbm, t2_v, sem_b)
    lanes = lax.iota(jnp.int32, 16)
    for l0 in range(L):
        lvt_v[pl.ds(16 * l0, 16)] = lax.rem(lanes + l0, L)
    cp_idx.wait()
    cp_t2.wait()

    def rg_body(rg, _):
        rv = lanes + rg * 16
        acc = jnp.zeros((16,), jnp.float32)
        for g in range(G):
            gv = jnp.full((16,), g, jnp.int32)
            for l0 in range(L):
                lv = lvt_v[pl.ds(16 * l0, 16)]
                iv = plsc.load_gather(idx_v, [rv, gv, lv])
                acc = acc + plsc.load_gather(t2_v, [gv, iv])
        out_v[pl.ds(rg * 16, 16)] = acc
        return 0

    lax.fori_loop(0, ROWS_PER_W // 16, rg_body, 0)
    pltpu.sync_copy(out_v, out_hbm.at[pl.ds(base_row, ROWS_PER_W)])


def kernel(x, table, W, b):
    w2 = W.reshape(G, D)
    t2 = pl.pallas_call(
        _tc_project,
        in_specs=[
            pl.BlockSpec(memory_space=pltpu.SMEM),
            pl.BlockSpec(memory_space=pltpu.VMEM),
            pl.BlockSpec(memory_space=pltpu.VMEM),
        ],
        out_shape=jax.ShapeDtypeStruct((G, V), jnp.float32),
    )(b, w2, table)

    sc = pl.kernel(
        _sc_pool,
        out_type=jax.ShapeDtypeStruct((B,), jnp.float32),
        mesh=plsc.VectorSubcoreMesh(core_axis_name="c", subcore_axis_name="s"),
        compiler_params=pltpu.CompilerParams(needs_layout_passes=False),
        scratch_types=[
            pltpu.VMEM((ROWS_PER_W, G, L), jnp.int32),
            pltpu.VMEM((G, V), jnp.float32),
            pltpu.VMEM((16 * L,), jnp.int32),
            pltpu.VMEM((ROWS_PER_W,), jnp.float32),
            pltpu.SemaphoreType.DMA,
            pltpu.SemaphoreType.DMA,
        ],
    )
    out = sc(x.astype(jnp.int32), t2)
    return out.reshape(B, 1)
